# baseline (speedup 1.0000x reference)
import functools

import jax
import jax.numpy as jnp
from jax import lax
from jax.experimental import pallas as pl
from jax.experimental.pallas import tpu as pltpu
from jax.experimental.pallas import tpu_sc as plsc

_B, _N, _K, _D = 2, 50000, 16, 16
_NK = _N * _K
_NW = 32
_ROWS_W = _NK // _NW
_CH = 1000
_NCHUNK = _ROWS_W // _CH

_RB = 6400
_NBLK = _RB // _K
_GRID_J = _NK // _RB


_NBT = 2560
_NPREP = -(-_N // _NBT)


def _prep_body(c_ref, f_ref, ct_ref, ft_ref):
    c = c_ref[0]
    ct_ref[0] = jnp.concatenate(
        [c, jnp.zeros((_NBT, 13), jnp.float32)], axis=1
    )
    eye = jnp.eye(16, dtype=jnp.float32)
    ft_ref[0] = lax.dot_general(
        f_ref[0], eye, (((0,), (0,)), ((), ())),
        preferred_element_type=jnp.float32,
    )


def _prep_pass(coords, feats):
    return pl.pallas_call(
        _prep_body,
        grid=(_B, _NPREP),
        in_specs=[
            pl.BlockSpec((1, _NBT, 3), lambda b, j: (b, j, 0)),
            pl.BlockSpec((1, 16, _NBT), lambda b, j: (b, 0, j)),
        ],
        out_specs=[
            pl.BlockSpec((1, _NBT, 16), lambda b, j: (b, j, 0)),
            pl.BlockSpec((1, _NBT, 16), lambda b, j: (b, j, 0)),
        ],
        out_shape=[
            jax.ShapeDtypeStruct((_B, _N, 16), jnp.float32),
            jax.ShapeDtypeStruct((_B, _N, 16), jnp.float32),
        ],
    )(coords, feats)


def _sc_gather(ct, ft, idx):
    mesh = plsc.VectorSubcoreMesh(core_axis_name="c", subcore_axis_name="s")

    @functools.partial(
        pl.kernel,
        mesh=mesh,
        out_type=[
            jax.ShapeDtypeStruct((_B, _NK, 16), jnp.float32),
            jax.ShapeDtypeStruct((_B, _NK, 16), jnp.float32),
        ],
        scratch_types=[
            pltpu.VMEM((_CH,), jnp.int32),
            pltpu.VMEM((_CH, 16), jnp.float32),
            pltpu.VMEM((_CH, 16), jnp.float32),
            pltpu.SemaphoreType.DMA,
        ],
        compiler_params=pltpu.CompilerParams(use_tc_tiling_on_sc=False),
    )
    def k(ct_hbm, ft_hbm, idx_hbm, gc_hbm, gf_hbm, idx_v, c_v, f_v, sem):
        wid = lax.axis_index("s") * 2 + lax.axis_index("c")
        for b in range(_B):
            for t in range(_NCHUNK):
                base = wid * _ROWS_W + t * _CH
                pltpu.sync_copy(idx_hbm.at[pl.ds(b * _NK + base, _CH)], idx_v)
                pltpu.async_copy(ct_hbm.at[b].at[idx_v], c_v, sem).wait()
                pltpu.sync_copy(c_v, gc_hbm.at[b, pl.ds(base, _CH)])
                pltpu.async_copy(ft_hbm.at[b].at[idx_v], f_v, sem).wait()
                pltpu.sync_copy(f_v, gf_hbm.at[b, pl.ds(base, _CH)])

    return k(ct, ft, idx)


def _x_block(cc, gc, wd, wcat, brow):
    ext = jnp.broadcast_to(
        cc.reshape(_NBLK, 1, 16), (_NBLK, _K, 16)
    ).reshape(_RB, 16)
    rp = ext - gc
    dist = jnp.sqrt(jnp.sum(rp * rp, axis=1, keepdims=True))
    rf = jnp.concatenate([rp, ext, gc], axis=1)
    x = lax.dot_general(
        rf, wcat, (((1,), (0,)), ((), ())), preferred_element_type=jnp.float32
    )
    return x + dist * wd + brow


def _stats_body(cc_ref, gc_ref, wd_ref, wc_ref, b_ref, out_ref, acc_ref):
    bi = pl.program_id(0)
    j = pl.program_id(1)

    @pl.when(jnp.logical_and(bi == 0, j == 0))
    def _():
        acc_ref[...] = jnp.zeros_like(acc_ref)

    x = _x_block(cc_ref[0], gc_ref[0], wd_ref[...], wc_ref[...], b_ref[...])
    acc_ref[0:1, :] += jnp.sum(x, axis=0, keepdims=True)
    acc_ref[1:2, :] += jnp.sum(x * x, axis=0, keepdims=True)

    @pl.when(jnp.logical_and(bi == _B - 1, j == _GRID_J - 1))
    def _():
        out_ref[...] = acc_ref[...]


def _out_body(cc_ref, gc_ref, gf_ref, wd_ref, wc_ref, b_ref, out_ref):
    x = _x_block(cc_ref[0], gc_ref[0], wd_ref[...], wc_ref[...], b_ref[...])
    x = jnp.maximum(x, 0.0)
    rows = jnp.concatenate([gf_ref[0], x], axis=1)
    eye = jnp.eye(32, dtype=jnp.float32)
    out_ref[0] = lax.dot_general(
        eye, rows, (((1,), (1,)), ((), ())), preferred_element_type=jnp.float32
    )


_SMALL_SPECS = [
    pl.BlockSpec((1, 16), lambda b, j: (0, 0)),
    pl.BlockSpec((48, 16), lambda b, j: (0, 0)),
    pl.BlockSpec((1, 16), lambda b, j: (0, 0)),
]


def _stats_pass(cc, gc, wd, wcat, brow):
    return pl.pallas_call(
        _stats_body,
        grid=(_B, _GRID_J),
        in_specs=[
            pl.BlockSpec((1, _NBLK, 16), lambda b, j: (b, j, 0)),
            pl.BlockSpec((1, _RB, 16), lambda b, j: (b, j, 0)),
        ] + _SMALL_SPECS,
        out_specs=pl.BlockSpec((2, 16), lambda b, j: (0, 0)),
        out_shape=jax.ShapeDtypeStruct((2, 16), jnp.float32),
        scratch_shapes=[pltpu.VMEM((2, 16), jnp.float32)],
    )(cc, gc, wd, wcat, brow)


def _out_pass(cc, gc, gf, wd, wcat, brow):
    return pl.pallas_call(
        _out_body,
        grid=(_B, _GRID_J),
        in_specs=[
            pl.BlockSpec((1, _NBLK, 16), lambda b, j: (b, j, 0)),
            pl.BlockSpec((1, _RB, 16), lambda b, j: (b, j, 0)),
            pl.BlockSpec((1, _RB, 16), lambda b, j: (b, j, 0)),
        ] + _SMALL_SPECS,
        out_specs=pl.BlockSpec((1, 32, _RB), lambda b, j: (b, 0, j)),
        out_shape=jax.ShapeDtypeStruct((_B, 32, _NK), jnp.float32),
    )(cc, gc, gf, wd, wcat, brow)


def kernel(coords, features, W, b, gamma, beta, neighbor_indices):
    ct, ft = _prep_pass(coords, features[..., 0])
    idx = neighbor_indices.reshape(_B * _NK)

    gc, gf = _sc_gather(ct, ft, idx)

    Wt = W.T.astype(jnp.float32)
    wd = Wt[0:1]
    z = jnp.zeros((13, 16), jnp.float32)
    wcat = jnp.concatenate(
        [Wt[1:4], z, Wt[4:7], z, Wt[7:10], z], axis=0
    )
    brow = b.reshape(1, 16).astype(jnp.float32)

    sums = _stats_pass(ct, gc, wd, wcat, brow)
    m = float(_B * _NK)
    mean = sums[0] / m
    var = sums[1] / m - mean * mean
    scale = gamma / jnp.sqrt(var + 1e-6)
    shift = beta - mean * scale
    wd2 = wd * scale[None, :]
    wcat2 = wcat * scale[None, :]
    b2 = brow * scale[None, :] + shift[None, :]

    out = _out_pass(ct, gc, gf, wd2, wcat2, b2)
    return out.reshape(_B, 2 * _D, _N, _K)


# speedup vs baseline: 1.0119x; 1.0119x over previous
import functools

import jax
import jax.numpy as jnp
from jax import lax
from jax.experimental import pallas as pl
from jax.experimental.pallas import tpu as pltpu
from jax.experimental.pallas import tpu_sc as plsc

_B, _N, _K, _D = 2, 50000, 16, 16
_NK = _N * _K
_NW = 32
_ROWS_W = _NK // _NW
_CH = 1000
_NCHUNK = _ROWS_W // _CH

_RB = 6400
_NBLK = _RB // _K
_GRID_J = _NK // _RB


_NBT = 2560
_NPREP = -(-_N // _NBT)


def _prep_body(c_ref, f_ref, ct_ref, ft_ref):
    c = c_ref[0]
    ct_ref[0] = jnp.concatenate(
        [c, jnp.zeros((_NBT, 13), jnp.float32)], axis=1
    )
    eye = jnp.eye(16, dtype=jnp.float32)
    ft_ref[0] = lax.dot_general(
        f_ref[0], eye, (((0,), (0,)), ((), ())),
        preferred_element_type=jnp.float32,
    )


def _prep_pass(coords, feats):
    return pl.pallas_call(
        _prep_body,
        grid=(_B, _NPREP),
        in_specs=[
            pl.BlockSpec((1, _NBT, 3), lambda b, j: (b, j, 0)),
            pl.BlockSpec((1, 16, _NBT), lambda b, j: (b, 0, j)),
        ],
        out_specs=[
            pl.BlockSpec((1, _NBT, 16), lambda b, j: (b, j, 0)),
            pl.BlockSpec((1, _NBT, 16), lambda b, j: (b, j, 0)),
        ],
        out_shape=[
            jax.ShapeDtypeStruct((_B, _N, 16), jnp.float32),
            jax.ShapeDtypeStruct((_B, _N, 16), jnp.float32),
        ],
    )(coords, feats)


def _sc_gather(ct, ft, idx):
    mesh = plsc.VectorSubcoreMesh(core_axis_name="c", subcore_axis_name="s")

    @functools.partial(
        pl.kernel,
        mesh=mesh,
        out_type=[
            jax.ShapeDtypeStruct((_B, _NK, 16), jnp.float32),
            jax.ShapeDtypeStruct((_B, _NK, 16), jnp.float32),
        ],
        scratch_types=[
            pltpu.VMEM((_CH,), jnp.int32),
            pltpu.VMEM((_CH, 16), jnp.float32),
            pltpu.VMEM((_CH, 16), jnp.float32),
            pltpu.SemaphoreType.DMA,
        ],
        compiler_params=pltpu.CompilerParams(use_tc_tiling_on_sc=False),
    )
    def k(ct_hbm, ft_hbm, idx_hbm, gc_hbm, gf_hbm, idx_v, c_v, f_v, sem):
        wid = lax.axis_index("s") * 2 + lax.axis_index("c")
        for b in range(_B):
            for t in range(_NCHUNK):
                base = wid * _ROWS_W + t * _CH
                pltpu.sync_copy(idx_hbm.at[pl.ds(b * _NK + base, _CH)], idx_v)
                pltpu.async_copy(ct_hbm.at[b].at[idx_v], c_v, sem).wait()
                pltpu.sync_copy(c_v, gc_hbm.at[b, pl.ds(base, _CH)])
                pltpu.async_copy(ft_hbm.at[b].at[idx_v], f_v, sem).wait()
                pltpu.sync_copy(f_v, gf_hbm.at[b, pl.ds(base, _CH)])

    return k(ct, ft, idx)


def _x_block(cc, gc, wd, wcat, brow):
    ext = jnp.broadcast_to(
        cc.reshape(_NBLK, 1, 16), (_NBLK, _K, 16)
    ).reshape(_RB, 16)
    rp = ext - gc
    dist = jnp.sqrt(jnp.sum(rp * rp, axis=1, keepdims=True))
    rf = jnp.concatenate([rp, ext, gc], axis=1)
    x = lax.dot_general(
        rf, wcat, (((1,), (0,)), ((), ())), preferred_element_type=jnp.float32
    )
    return x + dist * wd + brow


def _stats_body(cc_ref, gc_ref, wd_ref, wc_ref, b_ref, out_ref, acc_ref):
    bi = pl.program_id(0)
    j = pl.program_id(1)

    @pl.when(jnp.logical_and(bi == 0, j == 0))
    def _():
        acc_ref[...] = jnp.zeros_like(acc_ref)

    x = _x_block(cc_ref[0], gc_ref[0], wd_ref[...], wc_ref[...], b_ref[...])
    acc_ref[0:1, :] += jnp.sum(x, axis=0, keepdims=True)
    acc_ref[1:2, :] += jnp.sum(x * x, axis=0, keepdims=True)

    @pl.when(jnp.logical_and(bi == _B - 1, j == _GRID_J - 1))
    def _():
        out_ref[...] = acc_ref[...]


def _out_body(cc_ref, gc_ref, gf_ref, wd_ref, wc_ref, b_ref, out_ref):
    x = _x_block(cc_ref[0], gc_ref[0], wd_ref[...], wc_ref[...], b_ref[...])
    x = jnp.maximum(x, 0.0)
    rows = jnp.concatenate([gf_ref[0], x], axis=1)
    eye = jnp.eye(32, dtype=jnp.float32)
    cm = lax.dot_general(
        eye, rows, (((1,), (1,)), ((), ())), preferred_element_type=jnp.float32
    )
    out_ref[0] = cm.reshape(32, _NBLK, _K)


_SMALL_SPECS = [
    pl.BlockSpec((1, 16), lambda b, j: (0, 0)),
    pl.BlockSpec((48, 16), lambda b, j: (0, 0)),
    pl.BlockSpec((1, 16), lambda b, j: (0, 0)),
]


def _stats_pass(cc, gc, wd, wcat, brow):
    return pl.pallas_call(
        _stats_body,
        grid=(_B, _GRID_J),
        in_specs=[
            pl.BlockSpec((1, _NBLK, 16), lambda b, j: (b, j, 0)),
            pl.BlockSpec((1, _RB, 16), lambda b, j: (b, j, 0)),
        ] + _SMALL_SPECS,
        out_specs=pl.BlockSpec((2, 16), lambda b, j: (0, 0)),
        out_shape=jax.ShapeDtypeStruct((2, 16), jnp.float32),
        scratch_shapes=[pltpu.VMEM((2, 16), jnp.float32)],
    )(cc, gc, wd, wcat, brow)


def _out_pass(cc, gc, gf, wd, wcat, brow):
    return pl.pallas_call(
        _out_body,
        grid=(_B, _GRID_J),
        in_specs=[
            pl.BlockSpec((1, _NBLK, 16), lambda b, j: (b, j, 0)),
            pl.BlockSpec((1, _RB, 16), lambda b, j: (b, j, 0)),
            pl.BlockSpec((1, _RB, 16), lambda b, j: (b, j, 0)),
        ] + _SMALL_SPECS,
        out_specs=pl.BlockSpec((1, 32, _NBLK, _K), lambda b, j: (b, 0, j, 0)),
        out_shape=jax.ShapeDtypeStruct((_B, 32, _N, _K), jnp.float32),
    )(cc, gc, gf, wd, wcat, brow)


def kernel(coords, features, W, b, gamma, beta, neighbor_indices):
    ct, ft = _prep_pass(coords, features[..., 0])
    idx = neighbor_indices.reshape(_B * _NK)

    gc, gf = _sc_gather(ct, ft, idx)

    Wt = W.T.astype(jnp.float32)
    wd = Wt[0:1]
    z = jnp.zeros((13, 16), jnp.float32)
    wcat = jnp.concatenate(
        [Wt[1:4], z, Wt[4:7], z, Wt[7:10], z], axis=0
    )
    brow = b.reshape(1, 16).astype(jnp.float32)

    sums = _stats_pass(ct, gc, wd, wcat, brow)
    m = float(_B * _NK)
    mean = sums[0] / m
    var = sums[1] / m - mean * mean
    scale = gamma / jnp.sqrt(var + 1e-6)
    shift = beta - mean * scale
    wd2 = wd * scale[None, :]
    wcat2 = wcat * scale[None, :]
    b2 = brow * scale[None, :] + shift[None, :]

    return _out_pass(ct, gc, gf, wd2, wcat2, b2)


# speedup vs baseline: 1.0400x; 1.0278x over previous
import functools

import jax
import jax.numpy as jnp
from jax import lax
from jax.experimental import pallas as pl
from jax.experimental.pallas import tpu as pltpu
from jax.experimental.pallas import tpu_sc as plsc

_B, _N, _K, _D = 2, 50000, 16, 16
_NK = _N * _K
_NW = 32
_ROWS_W = _NK // _NW
_CH = 1000
_NCHUNK = _ROWS_W // _CH

_RB = 6400
_NBLK = _RB // _K
_GRID_J = _NK // _RB


_NBT = 2560
_NPREP = -(-_N // _NBT)


def _prep_body(c_ref, f_ref, ct_ref, ft_ref):
    c = c_ref[0]
    ct_ref[0] = jnp.concatenate(
        [c, jnp.zeros((_NBT, 13), jnp.float32)], axis=1
    )
    eye = jnp.eye(16, dtype=jnp.float32)
    ft_ref[0] = lax.dot_general(
        f_ref[0], eye, (((0,), (0,)), ((), ())),
        preferred_element_type=jnp.float32,
    )


def _prep_pass(coords, feats):
    return pl.pallas_call(
        _prep_body,
        grid=(_B, _NPREP),
        in_specs=[
            pl.BlockSpec((1, _NBT, 3), lambda b, j: (b, j, 0)),
            pl.BlockSpec((1, 16, _NBT), lambda b, j: (b, 0, j)),
        ],
        out_specs=[
            pl.BlockSpec((1, _NBT, 16), lambda b, j: (b, j, 0)),
            pl.BlockSpec((1, _NBT, 16), lambda b, j: (b, j, 0)),
        ],
        out_shape=[
            jax.ShapeDtypeStruct((_B, _N, 16), jnp.float32),
            jax.ShapeDtypeStruct((_B, _N, 16), jnp.float32),
        ],
    )(coords, feats)


def _sc_gather(ct, ft, idx):
    mesh = plsc.VectorSubcoreMesh(core_axis_name="c", subcore_axis_name="s")

    @functools.partial(
        pl.kernel,
        mesh=mesh,
        out_type=[
            jax.ShapeDtypeStruct((_B, _NK, 16), jnp.float32),
            jax.ShapeDtypeStruct((_B, _NK, 16), jnp.float32),
        ],
        scratch_types=[
            pltpu.VMEM((_CH,), jnp.int32),
            pltpu.VMEM((_CH, 16), jnp.float32),
            pltpu.VMEM((_CH, 16), jnp.float32),
            pltpu.SemaphoreType.DMA,
        ],
        compiler_params=pltpu.CompilerParams(use_tc_tiling_on_sc=False),
    )
    def k(ct_hbm, ft_hbm, idx_hbm, gc_hbm, gf_hbm, idx_v, c_v, f_v, sem):
        wid = lax.axis_index("s") * 2 + lax.axis_index("c")
        for b in range(_B):
            for t in range(_NCHUNK):
                base = wid * _ROWS_W + t * _CH
                pltpu.sync_copy(idx_hbm.at[pl.ds(b * _NK + base, _CH)], idx_v)
                pltpu.async_copy(ct_hbm.at[b].at[idx_v], c_v, sem).wait()
                pltpu.sync_copy(c_v, gc_hbm.at[b, pl.ds(base, _CH)])
                pltpu.async_copy(ft_hbm.at[b].at[idx_v], f_v, sem).wait()
                pltpu.sync_copy(f_v, gf_hbm.at[b, pl.ds(base, _CH)])

    return k(ct, ft, idx)


def _x_block(cc, gc, wd, wa, wb, brow):
    t = lax.dot_general(
        cc, wa, (((1,), (0,)), ((), ())), preferred_element_type=jnp.float32
    ) + brow
    tb = jnp.broadcast_to(
        t.reshape(_NBLK, 1, 16), (_NBLK, _K, 16)
    ).reshape(_RB, 16)
    ext = jnp.broadcast_to(
        cc.reshape(_NBLK, 1, 16), (_NBLK, _K, 16)
    ).reshape(_RB, 16)
    rp = ext - gc
    dist = jnp.sqrt(jnp.sum(rp * rp, axis=1, keepdims=True))
    x = lax.dot_general(
        gc, wb, (((1,), (0,)), ((), ())), preferred_element_type=jnp.float32
    )
    return x + dist * wd + tb


def _stats_body(cc_ref, gc_ref, wd_ref, wa_ref, wb_ref, b_ref, out_ref, acc_ref):
    bi = pl.program_id(0)
    j = pl.program_id(1)

    @pl.when(jnp.logical_and(bi == 0, j == 0))
    def _():
        acc_ref[...] = jnp.zeros_like(acc_ref)

    x = _x_block(cc_ref[0], gc_ref[0], wd_ref[...], wa_ref[...], wb_ref[...],
                 b_ref[...])
    acc_ref[0:1, :] += jnp.sum(x, axis=0, keepdims=True)
    acc_ref[1:2, :] += jnp.sum(x * x, axis=0, keepdims=True)

    @pl.when(jnp.logical_and(bi == _B - 1, j == _GRID_J - 1))
    def _():
        out_ref[...] = acc_ref[...]


def _out_body(cc_ref, gc_ref, gf_ref, wd_ref, wa_ref, wb_ref, b_ref, out_ref):
    x = _x_block(cc_ref[0], gc_ref[0], wd_ref[...], wa_ref[...], wb_ref[...],
                 b_ref[...])
    x = jnp.maximum(x, 0.0)
    rows = jnp.concatenate([gf_ref[0], x], axis=1)
    eye = jnp.eye(32, dtype=jnp.float32)
    cm = lax.dot_general(
        eye, rows, (((1,), (1,)), ((), ())), preferred_element_type=jnp.float32
    )
    out_ref[0] = cm.reshape(32, _NBLK, _K)


_SMALL_SPECS = [
    pl.BlockSpec((1, 16), lambda b, j: (0, 0)),
    pl.BlockSpec((16, 16), lambda b, j: (0, 0)),
    pl.BlockSpec((16, 16), lambda b, j: (0, 0)),
    pl.BlockSpec((1, 16), lambda b, j: (0, 0)),
]


def _stats_pass(cc, gc, wd, wa, wb, brow):
    return pl.pallas_call(
        _stats_body,
        grid=(_B, _GRID_J),
        in_specs=[
            pl.BlockSpec((1, _NBLK, 16), lambda b, j: (b, j, 0)),
            pl.BlockSpec((1, _RB, 16), lambda b, j: (b, j, 0)),
        ] + _SMALL_SPECS,
        out_specs=pl.BlockSpec((2, 16), lambda b, j: (0, 0)),
        out_shape=jax.ShapeDtypeStruct((2, 16), jnp.float32),
        scratch_shapes=[pltpu.VMEM((2, 16), jnp.float32)],
    )(cc, gc, wd, wa, wb, brow)


def _out_pass(cc, gc, gf, wd, wa, wb, brow):
    return pl.pallas_call(
        _out_body,
        grid=(_B, _GRID_J),
        in_specs=[
            pl.BlockSpec((1, _NBLK, 16), lambda b, j: (b, j, 0)),
            pl.BlockSpec((1, _RB, 16), lambda b, j: (b, j, 0)),
            pl.BlockSpec((1, _RB, 16), lambda b, j: (b, j, 0)),
        ] + _SMALL_SPECS,
        out_specs=pl.BlockSpec((1, 32, _NBLK, _K), lambda b, j: (b, 0, j, 0)),
        out_shape=jax.ShapeDtypeStruct((_B, 32, _N, _K), jnp.float32),
    )(cc, gc, gf, wd, wa, wb, brow)


def kernel(coords, features, W, b, gamma, beta, neighbor_indices):
    ct, ft = _prep_pass(coords, features[..., 0])
    idx = neighbor_indices.reshape(_B * _NK)

    gc, gf = _sc_gather(ct, ft, idx)

    Wt = W.T.astype(jnp.float32)
    wd = Wt[0:1]
    z = jnp.zeros((13, 16), jnp.float32)
    wrp = jnp.concatenate([Wt[1:4], z], axis=0)
    we = jnp.concatenate([Wt[4:7], z], axis=0)
    wn = jnp.concatenate([Wt[7:10], z], axis=0)
    wa = wrp + we
    wb = wn - wrp
    brow = b.reshape(1, 16).astype(jnp.float32)

    sums = _stats_pass(ct, gc, wd, wa, wb, brow)
    m = float(_B * _NK)
    mean = sums[0] / m
    var = sums[1] / m - mean * mean
    scale = gamma / jnp.sqrt(var + 1e-6)
    shift = beta - mean * scale
    wd2 = wd * scale[None, :]
    wa2 = wa * scale[None, :]
    wb2 = wb * scale[None, :]
    b2 = brow * scale[None, :] + shift[None, :]

    return _out_pass(ct, gc, gf, wd2, wa2, wb2, b2)
